# initial kernel scaffold (unmeasured)
import jax
import jax.numpy as jnp
from jax import lax
from jax.experimental import pallas as pl
from jax.experimental.pallas import tpu as pltpu

N_DEV = 4
N_PHASES = 6


def kernel(x, Win0, Wout0, Win1, Wout1, Win2, Wout2):
    b, d = x.shape

    def body(x_ref, win0, wout0, win1, wout1, win2, wout2, out_ref,
             xbuf0, xbuf1, xbuf2, pbuf0, pbuf1, pbuf2,
             psend0, psend1, psend2, send_sems, recv_sems):
        me = lax.axis_index("i")
        xbufs = [xbuf0, xbuf1, xbuf2]
        pbufs = [pbuf0, pbuf1, pbuf2]
        psends = [psend0, psend1, psend2]
        wins = [win0, win1, win2]
        wouts = [wout0, wout1, wout2]

        def exchange(phase, src_for, dst_buf):
            rdmas = []
            for off in range(1, N_DEV):
                tgt = (me + off) % N_DEV
                rdma = pltpu.make_async_remote_copy(
                    src_ref=src_for(tgt),
                    dst_ref=dst_buf.at[me],
                    send_sem=send_sems.at[phase, off - 1],
                    recv_sem=recv_sems.at[phase, off - 1],
                    device_id=(tgt,),
                    device_id_type=pl.DeviceIdType.MESH,
                )
                rdma.start()
                rdmas.append(rdma)
            for rdma in rdmas:
                rdma.wait()

        xbuf0[me] = x_ref[...]
        exchange(0, lambda tgt: x_ref, xbuf0)

        for l in range(3):
            X = xbufs[l][...].reshape(N_DEV * b, d)
            hdn = jnp.maximum(
                jnp.dot(X, wins[l][...], preferred_element_type=jnp.float32), 0.0
            )
            P = jnp.dot(hdn, wouts[l][...], preferred_element_type=jnp.float32)
            psends[l][...] = P.reshape(N_DEV, b, d)
            pbufs[l][me] = lax.dynamic_slice(P, (me * b, 0), (b, d))
            exchange(2 * l + 1, lambda tgt: psends[l].at[tgt], pbufs[l])
            reduced = jnp.sum(pbufs[l][...], axis=0)
            if l < 2:
                xbufs[l + 1][me] = reduced
                exchange(2 * l + 2, lambda tgt: xbufs[l + 1].at[me], xbufs[l + 1])
            else:
                out_ref[...] = reduced

    return pl.pallas_call(
        body,
        out_shape=jax.ShapeDtypeStruct((b, d), jnp.float32),
        in_specs=[pl.BlockSpec(memory_space=pltpu.VMEM)] * 7,
        out_specs=pl.BlockSpec(memory_space=pltpu.VMEM),
        scratch_shapes=(
            [pltpu.VMEM((N_DEV, b, d), jnp.float32)] * 3
            + [pltpu.VMEM((N_DEV, b, d), jnp.float32)] * 3
            + [pltpu.VMEM((N_DEV, b, d), jnp.float32)] * 3
            + [
                pltpu.SemaphoreType.DMA((N_PHASES, N_DEV - 1)),
                pltpu.SemaphoreType.DMA((N_PHASES, N_DEV - 1)),
            ]
        ),
        compiler_params=pltpu.CompilerParams(collective_id=0),
    )(x, Win0, Wout0, Win1, Wout1, Win2, Wout2)


# baseline (device time: 34506 ns/iter reference)
import jax
import jax.numpy as jnp
from jax import lax
from jax.experimental import pallas as pl
from jax.experimental.pallas import tpu as pltpu

N_DEV = 4
N_PHASES = 6


def kernel(x, Win0, Wout0, Win1, Wout1, Win2, Wout2):
    b, d = x.shape

    def body(x_ref, win0, wout0, win1, wout1, win2, wout2, out_ref,
             xbuf0, xbuf1, xbuf2, pbuf0, pbuf1, pbuf2,
             psend0, psend1, psend2, send_sems, recv_sems):
        me = lax.axis_index("i")
        xbufs = [xbuf0, xbuf1, xbuf2]
        pbufs = [pbuf0, pbuf1, pbuf2]
        psends = [psend0, psend1, psend2]
        wins = [win0, win1, win2]
        wouts = [wout0, wout1, wout2]

        def exchange(phase, src_for, dst_buf):
            rdmas = []
            for off in range(1, N_DEV):
                tgt = (me + off) % N_DEV
                rdma = pltpu.make_async_remote_copy(
                    src_ref=src_for(tgt),
                    dst_ref=dst_buf.at[me],
                    send_sem=send_sems.at[phase, off - 1],
                    recv_sem=recv_sems.at[phase, off - 1],
                    device_id=(tgt,),
                    device_id_type=pl.DeviceIdType.MESH,
                )
                rdma.start()
                rdmas.append(rdma)
            for rdma in rdmas:
                rdma.wait()

        xbuf0[me] = x_ref[...]
        exchange(0, lambda tgt: x_ref, xbuf0)

        for l in range(3):
            X = xbufs[l][...].reshape(N_DEV * b, d)
            hdn = jnp.maximum(
                jnp.dot(X, wins[l][...], preferred_element_type=jnp.float32), 0.0
            )
            P = jnp.dot(hdn, wouts[l][...], preferred_element_type=jnp.float32)
            psends[l][...] = P.reshape(N_DEV, b, d)
            pbufs[l][me] = psends[l][me]
            exchange(2 * l + 1, lambda tgt: psends[l].at[tgt], pbufs[l])
            reduced = jnp.sum(pbufs[l][...], axis=0)
            if l < 2:
                xbufs[l + 1][me] = reduced
                exchange(2 * l + 2, lambda tgt: xbufs[l + 1].at[me], xbufs[l + 1])
            else:
                out_ref[...] = reduced

    return pl.pallas_call(
        body,
        out_shape=jax.ShapeDtypeStruct((b, d), jnp.float32),
        in_specs=[pl.BlockSpec(memory_space=pltpu.VMEM)] * 7,
        out_specs=pl.BlockSpec(memory_space=pltpu.VMEM),
        scratch_shapes=(
            [pltpu.VMEM((N_DEV, b, d), jnp.float32)] * 3
            + [pltpu.VMEM((N_DEV, b, d), jnp.float32)] * 3
            + [pltpu.VMEM((N_DEV, b, d), jnp.float32)] * 3
            + [
                pltpu.SemaphoreType.DMA((N_PHASES, N_DEV - 1)),
                pltpu.SemaphoreType.DMA((N_PHASES, N_DEV - 1)),
            ]
        ),
    )(x, Win0, Wout0, Win1, Wout1, Win2, Wout2)


# device time: 32548 ns/iter; 1.0602x vs baseline; 1.0602x over previous
import jax
import jax.numpy as jnp
from jax import lax
from jax.experimental import pallas as pl
from jax.experimental.pallas import tpu as pltpu

N_DEV = 4
N_PHASES = 6
ORDER = (1, 3, 2)


def kernel(x, Win0, Wout0, Win1, Wout1, Win2, Wout2):
    b, d = x.shape

    def body(x_ref, win0, wout0, win1, wout1, win2, wout2, out_ref,
             xbuf0, xbuf1, xbuf2, pbuf0, pbuf1, pbuf2,
             psend0, psend1, psend2, send_sems, recv_sems):
        me = lax.axis_index("i")
        xbufs = [xbuf0, xbuf1, xbuf2]
        pbufs = [pbuf0, pbuf1, pbuf2]
        psends = [psend0, psend1, psend2]
        wins = [win0, win1, win2]
        wouts = [wout0, wout1, wout2]
        sends = []

        def send(phase, idx, src, dst_buf, tgt):
            rdma = pltpu.make_async_remote_copy(
                src_ref=src,
                dst_ref=dst_buf.at[me],
                send_sem=send_sems.at[phase, idx],
                recv_sem=recv_sems.at[phase, idx],
                device_id=(tgt,),
                device_id_type=pl.DeviceIdType.MESH,
            )
            rdma.start()
            sends.append(rdma)

        def wait_recv(phase, idx, buf):
            rdma = pltpu.make_async_remote_copy(
                src_ref=buf.at[me],
                dst_ref=buf.at[me],
                send_sem=send_sems.at[phase, idx],
                recv_sem=recv_sems.at[phase, idx],
                device_id=(me,),
                device_id_type=pl.DeviceIdType.MESH,
            )
            rdma.wait_recv()

        def mlp(xc, win, wout):
            h = jnp.maximum(jnp.dot(xc, win, preferred_element_type=jnp.float32), 0.0)
            return jnp.dot(h, wout, preferred_element_type=jnp.float32)

        xbuf0[me] = x_ref[...]
        for off in ORDER:
            send(0, off - 1, x_ref, xbuf0, (me + off) % N_DEV)

        for l in range(3):
            win = wins[l][...]
            wout = wouts[l][...]
            gphase = 2 * l
            sphase = 2 * l + 1
            pbufs[l][me] = mlp(xbufs[l][me], win, wout)
            for off in ORDER:
                src = (me - off) % N_DEV
                wait_recv(gphase, off - 1, xbufs[l])
                psends[l][src] = mlp(xbufs[l][src], win, wout)
                send(sphase, 3 - off, psends[l].at[src], pbufs[l], src)
            for off in ORDER:
                wait_recv(sphase, off - 1, pbufs[l])
            pv = pbufs[l][...]
            reduced = pv[0] + pv[1] + pv[2] + pv[3]
            if l < 2:
                xbufs[l + 1][me] = reduced
                for off in ORDER:
                    send(2 * l + 2, off - 1, xbufs[l + 1].at[me],
                         xbufs[l + 1], (me + off) % N_DEV)
            else:
                out_ref[...] = reduced

        for rdma in sends:
            rdma.wait_send()

    return pl.pallas_call(
        body,
        out_shape=jax.ShapeDtypeStruct((b, d), jnp.float32),
        in_specs=[pl.BlockSpec(memory_space=pltpu.VMEM)] * 7,
        out_specs=pl.BlockSpec(memory_space=pltpu.VMEM),
        scratch_shapes=(
            [pltpu.VMEM((N_DEV, b, d), jnp.float32)] * 3
            + [pltpu.VMEM((N_DEV, b, d), jnp.float32)] * 3
            + [pltpu.VMEM((N_DEV, b, d), jnp.float32)] * 3
            + [
                pltpu.SemaphoreType.DMA((N_PHASES, N_DEV - 1)),
                pltpu.SemaphoreType.DMA((N_PHASES, N_DEV - 1)),
            ]
        ),
    )(x, Win0, Wout0, Win1, Wout1, Win2, Wout2)
